# baseline (speedup 1.0000x reference)
import functools

import jax
import jax.numpy as jnp
from jax import lax
from jax.experimental import pallas as pl
from jax.experimental.pallas import tpu as pltpu
from jax.experimental.pallas import tpu_sc as plsc

BATCH = 4096
SEQ = 50
D = 64
B = BATCH * SEQ
NC, NS = 2, 16
NW = NC * NS
N_PER_W = B // NW
SUB = 128
K = 5
C = SUB * K
N_CHUNKS = N_PER_W // C
IROWS_PER_W = N_PER_W // SUB


def _lane_shuffle(v, perm):
    return v.at[perm].get(mode="promise_in_bounds")


def _normalize_rows(rows_v):
    lanes = lax.iota(jnp.int32, 16)
    perms = [lanes ^ k for k in (1, 2, 4, 8)]

    def body(i, carry):
        v0 = rows_v[i, pl.ds(0, 16)]
        v1 = rows_v[i, pl.ds(16, 16)]
        v2 = rows_v[i, pl.ds(32, 16)]
        v3 = rows_v[i, pl.ds(48, 16)]
        s = v0 * v0 + v1 * v1 + v2 * v2 + v3 * v3
        for p in perms:
            s = s + _lane_shuffle(s, p)
        ib = lax.bitcast_convert_type(s, jnp.int32)
        ib = jnp.int32(0x5F3759DF) - (ib >> 1)
        r = lax.bitcast_convert_type(ib, jnp.float32)
        r = r * (1.5 - 0.5 * s * r * r)
        r = r * (1.5 - 0.5 * s * r * r)
        r = r * (1.5 - 0.5 * s * r * r)
        inv = 1.0 / (s * r + 1e-8)
        rows_v[i, pl.ds(0, 16)] = v0 * inv
        rows_v[i, pl.ds(16, 16)] = v1 * inv
        rows_v[i, pl.ds(32, 16)] = v2 * inv
        rows_v[i, pl.ds(48, 16)] = v3 * inv
        return carry

    lax.fori_loop(0, C, body, 0)


def _sc_gather_norm(idx3d, table):
    mesh = plsc.VectorSubcoreMesh(core_axis_name="c", subcore_axis_name="s")

    @functools.partial(
        pl.kernel,
        mesh=mesh,
        out_type=jax.ShapeDtypeStruct((B, D), jnp.float32),
        compiler_params=pltpu.CompilerParams(use_tc_tiling_on_sc=False),
        scratch_types=[
            pltpu.VMEM((N_CHUNKS, 8, SUB), jnp.int32),
            pltpu.VMEM((C, D), jnp.float32),
            pltpu.SemaphoreType.DMA,
        ],
    )
    def k(idx_hbm, table_hbm, out_hbm, idx_v, rows_v, sem):
        wid = lax.axis_index("s") * NC + lax.axis_index("c")
        pltpu.sync_copy(idx_hbm.at[pl.ds(wid * N_CHUNKS, N_CHUNKS)], idx_v)

        def chunk_body(c, carry):
            row0 = wid * N_PER_W + c * C
            cps = [
                pltpu.async_copy(
                    table_hbm.at[idx_v.at[c, j]],
                    rows_v.at[pl.ds(j * SUB, SUB)],
                    sem,
                )
                for j in range(K)
            ]
            for cp in cps:
                cp.wait()
            _normalize_rows(rows_v)
            pltpu.sync_copy(rows_v, out_hbm.at[pl.ds(row0, C)])
            return carry

        lax.fori_loop(0, N_CHUNKS, chunk_body, 0)

    return k(idx3d, table)


def kernel(x, lengths, table):
    idx = x.astype(jnp.int32).reshape(NW * N_CHUNKS, C)
    idx = jnp.pad(idx, ((0, 0), (0, 8 * SUB - C)))
    idx3d = idx.reshape(NW * N_CHUNKS, 8, SUB)
    flat = _sc_gather_norm(idx3d, table)
    cap_emb = flat.reshape(BATCH, SEQ, D)
    cap_len = jnp.asarray(lengths, dtype=jnp.int32)
    return (cap_emb, cap_len)


# speedup vs baseline: 1.2715x; 1.2715x over previous
import functools

import jax
import jax.numpy as jnp
from jax import lax
from jax.experimental import pallas as pl
from jax.experimental.pallas import tpu as pltpu
from jax.experimental.pallas import tpu_sc as plsc

BATCH = 4096
SEQ = 50
D = 64
B = BATCH * SEQ
NC, NS = 2, 16
NW = NC * NS
N_PER_W = B // NW
SUB = 128
K = 5
C = SUB * K
N_CHUNKS = N_PER_W // C
IROWS_PER_W = N_PER_W // SUB


def _lane_shuffle(v, perm):
    return v.at[perm].get(mode="promise_in_bounds")


def _normalize_rows(rows_v):
    lanes = lax.iota(jnp.int32, 16)
    perms = [lanes ^ k for k in (1, 2, 4, 8)]

    @plsc.parallel_loop(0, C, 1, unroll=8)
    def body(i):
        v0 = rows_v[i, pl.ds(0, 16)]
        v1 = rows_v[i, pl.ds(16, 16)]
        v2 = rows_v[i, pl.ds(32, 16)]
        v3 = rows_v[i, pl.ds(48, 16)]
        s = v0 * v0 + v1 * v1 + v2 * v2 + v3 * v3
        for p in perms:
            s = s + _lane_shuffle(s, p)
        ib = lax.bitcast_convert_type(s, jnp.int32)
        ib = jnp.int32(0x5F3759DF) - (ib >> 1)
        r = lax.bitcast_convert_type(ib, jnp.float32)
        r = r * (1.5 - 0.5 * s * r * r)
        r = r * (1.5 - 0.5 * s * r * r)
        r = r * (1.5 - 0.5 * s * r * r)
        inv = 1.0 / (s * r + 1e-8)
        rows_v[i, pl.ds(0, 16)] = v0 * inv
        rows_v[i, pl.ds(16, 16)] = v1 * inv
        rows_v[i, pl.ds(32, 16)] = v2 * inv
        rows_v[i, pl.ds(48, 16)] = v3 * inv


def _sc_gather_norm(idx3d, table):
    mesh = plsc.VectorSubcoreMesh(core_axis_name="c", subcore_axis_name="s")

    @functools.partial(
        pl.kernel,
        mesh=mesh,
        out_type=jax.ShapeDtypeStruct((B, D), jnp.float32),
        compiler_params=pltpu.CompilerParams(use_tc_tiling_on_sc=False),
        scratch_types=[
            pltpu.VMEM((N_CHUNKS, 8, SUB), jnp.int32),
            pltpu.VMEM((2, C, D), jnp.float32),
            pltpu.SemaphoreType.DMA,
            pltpu.SemaphoreType.DMA,
            pltpu.SemaphoreType.DMA,
            pltpu.SemaphoreType.DMA,
        ],
    )
    def k(idx_hbm, table_hbm, out_hbm, idx_v, rows_v, g0, g1, w0, w1):
        gsem = (g0, g1)
        wsem = (w0, w1)
        wid = lax.axis_index("s") * NC + lax.axis_index("c")
        pltpu.sync_copy(idx_hbm.at[pl.ds(wid * N_CHUNKS, N_CHUNKS)], idx_v)

        def start_gathers(c, b):
            return [
                pltpu.async_copy(
                    table_hbm.at[idx_v.at[c, j]],
                    rows_v.at[b, pl.ds(j * SUB, SUB)],
                    gsem[b],
                )
                for j in range(K)
            ]

        gcps = {0: start_gathers(0, 0)}
        wcps = {}
        for c in range(N_CHUNKS):
            b, nb = c % 2, (c + 1) % 2
            if c + 1 < N_CHUNKS:
                if c - 1 in wcps:
                    wcps.pop(c - 1).wait()
                gcps[c + 1] = start_gathers(c + 1, nb)
            for cp in gcps.pop(c):
                cp.wait()
            _normalize_rows(rows_v.at[b])
            row0 = wid * N_PER_W + c * C
            wcps[c] = pltpu.async_copy(
                rows_v.at[b], out_hbm.at[pl.ds(row0, C)], wsem[b]
            )
        for cp in wcps.values():
            cp.wait()

    return k(idx3d, table)


def kernel(x, lengths, table):
    idx = x.astype(jnp.int32).reshape(NW * N_CHUNKS, C)
    idx = jnp.pad(idx, ((0, 0), (0, 8 * SUB - C)))
    idx3d = idx.reshape(NW * N_CHUNKS, 8, SUB)
    flat = _sc_gather_norm(idx3d, table)
    cap_emb = flat.reshape(BATCH, SEQ, D)
    cap_len = jnp.asarray(lengths, dtype=jnp.int32)
    return (cap_emb, cap_len)


# speedup vs baseline: 1.3078x; 1.0286x over previous
import functools

import jax
import jax.numpy as jnp
from jax import lax
from jax.experimental import pallas as pl
from jax.experimental.pallas import tpu as pltpu
from jax.experimental.pallas import tpu_sc as plsc

BATCH = 4096
SEQ = 50
D = 64
NC, NS = 2, 16
NW = NC * NS
B_PER_W = BATCH // NW
CB = 16
N_CHUNKS = B_PER_W // CB
ROWS = CB * SEQ


def _lane_shuffle(v, perm):
    return v.at[perm].get(mode="promise_in_bounds")


def _normalize_rows(rows_v):
    lanes = lax.iota(jnp.int32, 16)
    perms = [lanes ^ k for k in (1, 2, 4, 8)]

    @plsc.parallel_loop(0, ROWS, 1, unroll=8)
    def body(i):
        bi = i // SEQ
        si = i % SEQ
        v0 = rows_v[bi, si, pl.ds(0, 16)]
        v1 = rows_v[bi, si, pl.ds(16, 16)]
        v2 = rows_v[bi, si, pl.ds(32, 16)]
        v3 = rows_v[bi, si, pl.ds(48, 16)]
        s = v0 * v0 + v1 * v1 + v2 * v2 + v3 * v3
        for p in perms:
            s = s + _lane_shuffle(s, p)
        ib = lax.bitcast_convert_type(s, jnp.int32)
        ib = jnp.int32(0x5F3759DF) - (ib >> 1)
        r = lax.bitcast_convert_type(ib, jnp.float32)
        r = r * (1.5 - 0.5 * s * r * r)
        inv = r * (1.5 - 0.5 * s * r * r)
        rows_v[bi, si, pl.ds(0, 16)] = v0 * inv
        rows_v[bi, si, pl.ds(16, 16)] = v1 * inv
        rows_v[bi, si, pl.ds(32, 16)] = v2 * inv
        rows_v[bi, si, pl.ds(48, 16)] = v3 * inv


def _sc_gather_norm(idx, table):
    mesh = plsc.VectorSubcoreMesh(core_axis_name="c", subcore_axis_name="s")

    @functools.partial(
        pl.kernel,
        mesh=mesh,
        out_type=jax.ShapeDtypeStruct((BATCH, SEQ, D), jnp.float32),
        compiler_params=pltpu.CompilerParams(use_tc_tiling_on_sc=False),
        scratch_types=[
            pltpu.VMEM((B_PER_W, SEQ), jnp.int32),
            pltpu.VMEM((CB, SEQ, D), jnp.float32),
            pltpu.VMEM((CB, SEQ, D), jnp.float32),
            pltpu.SemaphoreType.DMA,
            pltpu.SemaphoreType.DMA,
            pltpu.SemaphoreType.DMA,
            pltpu.SemaphoreType.DMA,
        ],
    )
    def k(idx_hbm, table_hbm, out_hbm, idx_v, rows0, rows1, g0, g1, w0, w1):
        rbuf = (rows0, rows1)
        gsem = (g0, g1)
        wsem = (w0, w1)
        wid = lax.axis_index("s") * NC + lax.axis_index("c")
        batch0 = wid * B_PER_W
        pltpu.sync_copy(idx_hbm.at[pl.ds(batch0, B_PER_W)], idx_v)

        def start_gathers(c, b):
            return [
                pltpu.async_copy(
                    table_hbm.at[idx_v.at[c * CB + j]],
                    rbuf[b].at[j],
                    gsem[b],
                )
                for j in range(CB)
            ]

        gcps = {0: start_gathers(0, 0)}
        wcps = {}
        for c in range(N_CHUNKS):
            b, nb = c % 2, (c + 1) % 2
            if c + 1 < N_CHUNKS:
                if c - 1 in wcps:
                    wcps.pop(c - 1).wait()
                gcps[c + 1] = start_gathers(c + 1, nb)
            for cp in gcps.pop(c):
                cp.wait()
            _normalize_rows(rbuf[b])
            wcps[c] = pltpu.async_copy(
                rbuf[b],
                out_hbm.at[pl.ds(batch0 + c * CB, CB)],
                wsem[b],
            )
        for cp in wcps.values():
            cp.wait()

    return k(idx, table)


def kernel(x, lengths, table):
    cap_emb = _sc_gather_norm(x.astype(jnp.int32), table)
    cap_len = jnp.asarray(lengths, dtype=jnp.int32)
    return (cap_emb, cap_len)
